# baseline (speedup 1.0000x reference)
import dataclasses
import functools

import jax
import jax.numpy as jnp
from jax import lax
from jax.experimental import pallas as pl
from jax.experimental.pallas import tpu as pltpu
from jax.experimental.pallas import tpu_sc as plsc

CONTEXT = 16
VOCAB = 256
EMBED = 6
HIDDEN = 64
NOUT = 256
NFEAT = CONTEXT * EMBED

NUM_CORES = 2
NUM_SUBCORES = 16
NW = NUM_CORES * NUM_SUBCORES
LANES = 16


def _sc_gather_body(emb_hbm, x_hbm, out_hbm, emb_v, xv, h0t_v, sem):
    bpw = h0t_v.shape[1]
    wid = lax.axis_index("s") * NUM_CORES + lax.axis_index("c")
    pltpu.sync_copy(emb_hbm, emb_v)
    pltpu.sync_copy(x_hbm.at[pl.ds(wid * bpw * CONTEXT, bpw * CONTEXT)], xv)
    row0 = lax.iota(jnp.int32, LANES) * CONTEXT

    TCH = 4
    copies = []
    for t0 in range(0, CONTEXT, TCH):

        @plsc.parallel_loop(0, bpw, step=LANES, unroll=8)
        def _(b, t0=t0):
            for t in range(t0, t0 + TCH):
                xi = plsc.load_gather(xv, [row0 + (b * CONTEXT + t)])
                addr = xi * EMBED
                for d in range(EMBED):
                    v = plsc.load_gather(emb_v, [addr + d] if d else [addr])
                    h0t_v[t * EMBED + d, pl.ds(b, LANES)] = v

        copies.append(pltpu.async_copy(
            h0t_v.at[pl.ds(t0 * EMBED, TCH * EMBED)],
            out_hbm.at[wid, pl.ds(t0 * EMBED, TCH * EMBED)], sem))
    for c in copies:
        c.wait()


def _mlp_body(h0t_ref, w1t_ref, b1_ref, w2t_ref, b2_ref, out_ref):
    nw_blk, _, bpw = h0t_ref.shape
    for k in range(nw_blk):
        h0t = h0t_ref[k].astype(jnp.bfloat16)
        h1 = lax.dot_general(
            h0t, w1t_ref[...], (((0,), (0,)), ((), ())),
            preferred_element_type=jnp.float32,
        )
        h1 = jnp.maximum(h1 + b1_ref[...], 0.0).astype(jnp.bfloat16)
        out_ref[pl.ds(k * bpw, bpw), :] = lax.dot_general(
            h1, w2t_ref[...], (((1,), (0,)), ((), ())),
            preferred_element_type=jnp.float32,
        ) + b2_ref[...]


def kernel(x, embed, W1, b1, W2, b2):
    batch = x.shape[0]
    bpw = batch // NW
    x = x.astype(jnp.int32)
    emb_flat = embed.reshape(VOCAB * EMBED)

    cp = pltpu.CompilerParams()
    if "needs_layout_passes" in pltpu.CompilerParams.__dataclass_fields__:
        cp = dataclasses.replace(cp, needs_layout_passes=False)
    mesh = plsc.VectorSubcoreMesh(core_axis_name="c", subcore_axis_name="s")
    sc_gather = functools.partial(
        pl.kernel,
        mesh=mesh,
        compiler_params=cp,
        out_type=jax.ShapeDtypeStruct((NW, NFEAT, bpw), jnp.float32),
        scratch_types=[
            pltpu.VMEM((VOCAB * EMBED,), jnp.float32),
            pltpu.VMEM((bpw * CONTEXT,), jnp.int32),
            pltpu.VMEM((NFEAT, bpw), jnp.float32),
            pltpu.SemaphoreType.DMA,
        ],
    )(_sc_gather_body)
    h0t = sc_gather(emb_flat, x.reshape(-1))

    w1t_bf = W1.T.astype(jnp.bfloat16)
    w2t_bf = W2.T.astype(jnp.bfloat16)
    b1_row = b1.reshape(1, HIDDEN)
    b2_row = b2.reshape(1, NOUT)

    wpb = 16
    out = pl.pallas_call(
        _mlp_body,
        grid=(NW // wpb,),
        in_specs=[
            pl.BlockSpec((wpb, NFEAT, bpw), lambda i: (i, 0, 0)),
            pl.BlockSpec((NFEAT, HIDDEN), lambda i: (0, 0)),
            pl.BlockSpec((1, HIDDEN), lambda i: (0, 0)),
            pl.BlockSpec((HIDDEN, NOUT), lambda i: (0, 0)),
            pl.BlockSpec((1, NOUT), lambda i: (0, 0)),
        ],
        out_specs=pl.BlockSpec((wpb * bpw, NOUT), lambda i: (i, 0)),
        out_shape=jax.ShapeDtypeStruct((batch, NOUT), jnp.float32),
    )(h0t, w1t_bf, b1_row, w2t_bf, b2_row)
    return out


# speedup vs baseline: 1.3120x; 1.3120x over previous
import dataclasses
import functools

import jax
import jax.numpy as jnp
from jax import lax
from jax.experimental import pallas as pl
from jax.experimental.pallas import tpu as pltpu
from jax.experimental.pallas import tpu_sc as plsc

CONTEXT = 16
VOCAB = 256
EMBED = 6
HIDDEN = 64
NOUT = 256
NFEAT = CONTEXT * EMBED

NUM_CORES = 2
NUM_SUBCORES = 16
NW = NUM_CORES * NUM_SUBCORES
LANES = 16


def _sc_gather_body(emb_hbm, xprep_hbm, out_hbm, emb_v, xv, h0t_v, sem):
    bpw = h0t_v.shape[1]
    wid = lax.axis_index("s") * NUM_CORES + lax.axis_index("c")
    pltpu.sync_copy(emb_hbm, emb_v)
    pltpu.sync_copy(xprep_hbm.at[pl.ds(wid * bpw * CONTEXT, bpw * CONTEXT)], xv)

    TCH = 4
    copies = []
    for t0 in range(0, CONTEXT, TCH):

        @plsc.parallel_loop(0, bpw, step=LANES, unroll=8)
        def _(b, t0=t0):
            for t in range(t0, t0 + TCH):
                addr = xv[pl.ds(t * bpw + b, LANES)]
                for d in range(EMBED):
                    v = plsc.load_gather(emb_v, [addr + d] if d else [addr])
                    h0t_v[t * EMBED + d, pl.ds(b, LANES)] = v

        copies.append(pltpu.async_copy(
            h0t_v.at[pl.ds(t0 * EMBED, TCH * EMBED)],
            out_hbm.at[wid, pl.ds(t0 * EMBED, TCH * EMBED)], sem))
    for c in copies:
        c.wait()


def _mlp_body(h0t_ref, w1t_ref, b1_ref, w2t_ref, b2_ref, out_ref):
    nw_blk, _, bpw = h0t_ref.shape
    for k in range(nw_blk):
        h0t = h0t_ref[k].astype(jnp.bfloat16)
        h1 = lax.dot_general(
            h0t, w1t_ref[...], (((0,), (0,)), ((), ())),
            preferred_element_type=jnp.float32,
        )
        h1 = jnp.maximum(h1 + b1_ref[...], 0.0).astype(jnp.bfloat16)
        out_ref[pl.ds(k * bpw, bpw), :] = lax.dot_general(
            h1, w2t_ref[...], (((1,), (0,)), ((), ())),
            preferred_element_type=jnp.float32,
        ) + b2_ref[...]


def kernel(x, embed, W1, b1, W2, b2):
    batch = x.shape[0]
    bpw = batch // NW
    x = x.astype(jnp.int32)
    xprep = (x * EMBED).reshape(NW, bpw, CONTEXT).transpose(0, 2, 1).reshape(-1)
    emb_flat = embed.reshape(VOCAB * EMBED)

    cp = pltpu.CompilerParams()
    if "needs_layout_passes" in pltpu.CompilerParams.__dataclass_fields__:
        cp = dataclasses.replace(cp, needs_layout_passes=False)
    mesh = plsc.VectorSubcoreMesh(core_axis_name="c", subcore_axis_name="s")
    sc_gather = functools.partial(
        pl.kernel,
        mesh=mesh,
        compiler_params=cp,
        out_type=jax.ShapeDtypeStruct((NW, NFEAT, bpw), jnp.float32),
        scratch_types=[
            pltpu.VMEM((VOCAB * EMBED,), jnp.float32),
            pltpu.VMEM((bpw * CONTEXT,), jnp.int32),
            pltpu.VMEM((NFEAT, bpw), jnp.float32),
            pltpu.SemaphoreType.DMA,
        ],
    )(_sc_gather_body)
    h0t = sc_gather(emb_flat, xprep)

    w1t_bf = W1.T.astype(jnp.bfloat16)
    w2t_bf = W2.T.astype(jnp.bfloat16)
    b1_row = b1.reshape(1, HIDDEN)
    b2_row = b2.reshape(1, NOUT)

    wpb = 16
    out = pl.pallas_call(
        _mlp_body,
        grid=(NW // wpb,),
        in_specs=[
            pl.BlockSpec((wpb, NFEAT, bpw), lambda i: (i, 0, 0)),
            pl.BlockSpec((NFEAT, HIDDEN), lambda i: (0, 0)),
            pl.BlockSpec((1, HIDDEN), lambda i: (0, 0)),
            pl.BlockSpec((HIDDEN, NOUT), lambda i: (0, 0)),
            pl.BlockSpec((1, NOUT), lambda i: (0, 0)),
        ],
        out_specs=pl.BlockSpec((wpb * bpw, NOUT), lambda i: (i, 0)),
        out_shape=jax.ShapeDtypeStruct((batch, NOUT), jnp.float32),
    )(h0t, w1t_bf, b1_row, w2t_bf, b2_row)
    return out
